# baseline (speedup 1.0000x reference)
import jax
import jax.numpy as jnp
from jax.experimental import pallas as pl
from jax.experimental.pallas import tpu as pltpu

_ROWS = 16384
_D = 2048
_TAIL = 256
_RING_ROWS = _ROWS - _TAIL
_NCHUNK = 16
_CHUNK = _RING_ROWS // _NCHUNK
_NBUF = 3


def _copy_body(x_ref, o_ref, *rest):
    bufs = rest[:_NBUF]
    rsems = rest[_NBUF:2 * _NBUF]
    wsems = rest[2 * _NBUF:3 * _NBUF]
    tsem = rest[3 * _NBUF]
    tail = pltpu.make_async_copy(
        x_ref.at[pl.ds(_RING_ROWS, _TAIL)],
        o_ref.at[pl.ds(_RING_ROWS, _TAIL)], tsem)
    tail.start()
    reads = [None] * _NBUF
    writes = [None] * _NBUF
    for g in range(_NBUF - 1):
        reads[g] = pltpu.make_async_copy(
            x_ref.at[pl.ds(g * _CHUNK, _CHUNK)], bufs[g], rsems[g])
        reads[g].start()
    for g in range(_NCHUNK):
        b = g % _NBUF
        reads[b].wait()
        writes[b] = pltpu.make_async_copy(
            bufs[b], o_ref.at[pl.ds(g * _CHUNK, _CHUNK)], wsems[b])
        writes[b].start()
        nxt = g + _NBUF - 1
        if nxt < _NCHUNK:
            nb = nxt % _NBUF
            if writes[nb] is not None:
                writes[nb].wait()
            reads[nb] = pltpu.make_async_copy(
                x_ref.at[pl.ds(nxt * _CHUNK, _CHUNK)], bufs[nb], rsems[nb])
            reads[nb].start()
    for b in range(_NBUF):
        if writes[b] is not None:
            writes[b].wait()
    tail.wait()


def kernel(x, rel_pos_bias):
    del rel_pos_bias
    b, s, d = x.shape
    x2 = x.reshape(b * s, d)
    out = pl.pallas_call(
        _copy_body,
        out_shape=jax.ShapeDtypeStruct((b * s, d), x.dtype),
        in_specs=[pl.BlockSpec(memory_space=pl.ANY)],
        out_specs=pl.BlockSpec(memory_space=pl.ANY),
        scratch_shapes=(
            [pltpu.VMEM((_CHUNK, _D), jnp.float32)] * _NBUF
            + [pltpu.SemaphoreType.DMA] * (2 * _NBUF + 1)
        ),
    )(x2)
    return out.reshape(b, s, d)


# speedup vs baseline: 1.0055x; 1.0055x over previous
import jax
import jax.numpy as jnp
from jax.experimental import pallas as pl
from jax.experimental.pallas import tpu as pltpu

_ROWS = 16384
_D = 2048
_CHUNK = 2048
_NCHUNK = _ROWS // _CHUNK
_NBUF = 3


def _copy_body(x_ref, o_ref, *rest):
    bufs = rest[:_NBUF]
    rsems = rest[_NBUF:2 * _NBUF]
    wsems = rest[2 * _NBUF:3 * _NBUF]
    reads = [None] * _NBUF
    writes = [None] * _NBUF
    for g in range(_NBUF - 1):
        reads[g] = pltpu.make_async_copy(
            x_ref.at[pl.ds(g * _CHUNK, _CHUNK)], bufs[g], rsems[g])
        reads[g].start()
    for g in range(_NCHUNK):
        b = g % _NBUF
        reads[b].wait()
        writes[b] = pltpu.make_async_copy(
            bufs[b], o_ref.at[pl.ds(g * _CHUNK, _CHUNK)], wsems[b])
        writes[b].start()
        nxt = g + _NBUF - 1
        if nxt < _NCHUNK:
            nb = nxt % _NBUF
            if writes[nb] is not None:
                writes[nb].wait()
            reads[nb] = pltpu.make_async_copy(
                x_ref.at[pl.ds(nxt * _CHUNK, _CHUNK)], bufs[nb], rsems[nb])
            reads[nb].start()
    for b in range(_NBUF):
        if writes[b] is not None:
            writes[b].wait()


def kernel(x, rel_pos_bias):
    del rel_pos_bias
    b, s, d = x.shape
    x2 = x.reshape(b * s, d)
    out = pl.pallas_call(
        _copy_body,
        out_shape=jax.ShapeDtypeStruct((b * s, d), x.dtype),
        in_specs=[pl.BlockSpec(memory_space=pl.ANY)],
        out_specs=pl.BlockSpec(memory_space=pl.ANY),
        scratch_shapes=(
            [pltpu.VMEM((_CHUNK, _D), jnp.float32)] * _NBUF
            + [pltpu.SemaphoreType.DMA] * (2 * _NBUF)
        ),
    )(x2)
    return out.reshape(b, s, d)
